# baseline (speedup 1.0000x reference)
import functools

import jax
import jax.numpy as jnp
from jax import lax
from jax.experimental import pallas as pl
from jax.experimental.pallas import tpu as pltpu
from jax.experimental.pallas import tpu_sc as plsc

NU = 50000
NR = 50000
N = 50000
E = 500000
EL = 100000
D = 128
NCHUNK = 4
CW = 32
NSENT = 48
NROW = N + NSENT

NTILE = 16
EPAD = 524288
EB = 1024
NB = 32
TPE = EB * NB

ELP = 131072
LNB = 4

RPT = NROW // NTILE

BLK = 1000

_mesh = plsc.VectorSubcoreMesh(core_axis_name="c", subcore_axis_name="s")


def _f32(shape):
    return jax.ShapeDtypeStruct(shape, jnp.float32)


def _make_segsum(do_counts):
  out_t = (_f32((NROW, D)), _f32((NROW, D)))
  if do_counts:
    out_t = out_t + (_f32((NROW, 128)),)

  @functools.partial(
      pl.kernel,
      out_type=out_t,
      mesh=_mesh,
      scratch_types=[
          pltpu.VMEM_SHARED((NROW, CW), jnp.float32),
          pltpu.VMEM((32, 128), jnp.int32),
          pltpu.VMEM((32, 128), jnp.int32),
          pltpu.VMEM((512, CW), jnp.float32),
          pltpu.SemaphoreType.DMA,
          pltpu.SemaphoreType.DMA,
      ],
      compiler_params=pltpu.CompilerParams(use_tc_tiling_on_sc=False),
  )
  def _segsum_sc(tbl_r, tbl_u, g_ur, s_ur, g_ru, s_ru, zeros_hbm, *rest):
    if do_counts:
        ones_hbm, agg_r, agg_u, cnt_out = rest[:4]
        acc, gidxb, sidxb, rows_v, semg, sems = rest[4:]
    else:
        agg_r, agg_u = rest[:2]
        acc, gidxb, sidxb, rows_v, semg, sems = rest[2:]
    cid = lax.axis_index("c")
    sid = lax.axis_index("s")
    base_r = sid * (TPE // 128)

    def region(s):
        return rows_v.at[pl.ds((s % 4) * 128, 128)]

    for direction in range(2):
        tbl = (tbl_r, tbl_u)[direction]
        gi = (g_ur, g_ru)[direction]
        si = (s_ur, s_ru)[direction]
        out = (agg_r, agg_u)[direction]
        for j in range(2):
            chunk = cid * 2 + j
            gic = gi.at[chunk]
            pltpu.sync_copy(zeros_hbm.at[pl.ds(sid * RPT, RPT)],
                            acc.at[pl.ds(sid * RPT, RPT)])
            plsc.subcore_barrier()

            def batch(b, carry):
                roff = base_r + b * 32
                pltpu.sync_copy(gic.at[pl.ds(roff, 32)], gidxb)
                pltpu.sync_copy(si.at[pl.ds(roff, 32)], sidxb)
                gat = [None] * 32
                scat = [None] * 32
                for t in range(32):
                    if t >= 4:
                        scat[t - 4].wait()
                    else:
                        @pl.when(b > 0)
                        def _(t=t):
                            pltpu.make_async_copy(
                                region(t), acc.at[sidxb.at[t]], sems).wait()
                    gat[t] = pltpu.async_copy(tbl.at[gidxb.at[t]],
                                              region(t), semg)
                    if t >= 2:
                        gat[t - 2].wait()
                        scat[t - 2] = pltpu.async_copy(
                            region(t - 2), acc.at[sidxb.at[t - 2]], sems,
                            add=True)
                for t in (30, 31):
                    gat[t].wait()
                    scat[t] = pltpu.async_copy(region(t), acc.at[sidxb.at[t]],
                                               sems, add=True)
                return carry

            lax.fori_loop(0, NB // 4, batch, 0)
            for t in range(4):
                pltpu.make_async_copy(region(t), acc.at[sidxb.at[t]],
                                      sems).wait()
            plsc.subcore_barrier()
            pltpu.sync_copy(acc.at[pl.ds(sid * RPT, RPT)],
                            out.at[pl.ds(sid * RPT, RPT),
                                   pl.ds(chunk * CW, CW)])
            plsc.subcore_barrier()

    if do_counts:
        pltpu.sync_copy(ones_hbm, rows_v.at[pl.ds(0, 128)])
        pltpu.sync_copy(zeros_hbm.at[pl.ds(sid * RPT, RPT)],
                        acc.at[pl.ds(sid * RPT, RPT)])
        plsc.subcore_barrier()

        def hist(si):
            def hblk(b, carry):
                pltpu.sync_copy(si.at[pl.ds(base_r + b * 32, 32)], sidxb)
                for g in range(4):
                    scs = [
                        pltpu.async_copy(rows_v.at[pl.ds(0, 128)],
                                         acc.at[sidxb.at[g * 8 + t]], sems,
                                         add=True)
                        for t in range(8)
                    ]
                    for dsc in scs:
                        dsc.wait()
                return carry
            lax.fori_loop(0, NB // 4, hblk, 0)

        @pl.when(cid == 0)
        def _():
            hist(s_ur)

        @pl.when(cid == 1)
        def _():
            hist(s_ru)

        plsc.subcore_barrier()
        pltpu.sync_copy(acc.at[pl.ds(sid * RPT, RPT)],
                        cnt_out.at[pl.ds(sid * RPT, RPT),
                                   pl.ds(cid * 32, 32)])

  return _segsum_sc


_segsum_cnt = _make_segsum(True)
_segsum_plain = _make_segsum(False)


@functools.partial(
    pl.kernel,
    out_type=(_f32((ELP, D)), _f32((ELP, D))),
    mesh=_mesh,
    scratch_types=[
        pltpu.VMEM((8, 128), jnp.int32),
        pltpu.VMEM((512, D), jnp.float32),
        pltpu.SemaphoreType.DMA,
        pltpu.SemaphoreType.DMA,
    ],
    compiler_params=pltpu.CompilerParams(use_tc_tiling_on_sc=True),
)
def _decoder_sc(u2, r2, lsrc2, ldst2, z1, z2, idx_v, rows_v, semg, semo):
    cid = lax.axis_index("c")
    sid = lax.axis_index("s")
    wid = sid * 2 + cid
    per_tile = ELP // 32

    def buf(u):
        return rows_v.at[pl.ds((u % 2) * 256, 256)]

    for direction in range(2):
        tbl = (u2, r2)[direction]
        li = (lsrc2, ldst2)[direction]
        out = (z1, z2)[direction]

        def batch(b, carry):
            roff = wid * (per_tile // 128) + b * 8
            pltpu.sync_copy(li.at[pl.ds(roff, 8)], idx_v)
            ocp = [None] * 4
            for u in range(4):
                obase = wid * per_tile + b * 1024 + u * 256
                if u >= 2:
                    ocp[u - 2].wait()
                else:
                    @pl.when(b > 0)
                    def _(u=u, obase=obase):
                        pltpu.make_async_copy(
                            buf(u), out.at[pl.ds(obase, 256)], semo).wait()
                gats = [
                    pltpu.async_copy(tbl.at[idx_v.at[u * 2 + q]],
                                     buf(u).at[pl.ds(q * 128, 128)], semg)
                    for q in range(2)
                ]
                for dsc in gats:
                    dsc.wait()
                ocp[u] = pltpu.async_copy(buf(u), out.at[pl.ds(obase, 256)],
                                          semo)
            return carry

        lax.fori_loop(0, LNB, batch, 0)
        for u in range(2):
            pltpu.make_async_copy(buf(u), out.at[pl.ds(wid * per_tile, 256)],
                                  semo).wait()


def _enc_body(xu_ref, xr_ref, wut_ref, wrt_ref, bu_ref, br_ref,
              hu_ref, hr_ref):
    hu_ref[...] = jnp.dot(xu_ref[...], wut_ref[...],
                          preferred_element_type=jnp.float32) + bu_ref[...]
    hr_ref[...] = jnp.dot(xr_ref[...], wrt_ref[...],
                          preferred_element_type=jnp.float32) + br_ref[...]


def _encoder(xu, xr, wut, wrt, bu, br):
    grid = (N // BLK,)
    row = pl.BlockSpec((BLK, D), lambda i: (i, 0))
    full = pl.BlockSpec((D, D), lambda i: (0, 0))
    bias = pl.BlockSpec((1, D), lambda i: (0, 0))
    return pl.pallas_call(
        _enc_body,
        grid=grid,
        in_specs=[row, row, full, full, bias, bias],
        out_specs=[row, row],
        out_shape=[_f32((N, D)), _f32((N, D))],
    )(xu, xr, wut, wrt, bu, br)


def _combine_body(relu, coff, agg_ref, cnt_ref, h_ref, wlt_ref, wrt_ref,
                  b_ref, out_ref):
    inv = 1.0 / jnp.maximum(cnt_ref[:, coff:coff + 1], 1.0)
    acc = jnp.dot(h_ref[...], wrt_ref[...],
                  preferred_element_type=jnp.float32) + b_ref[...]
    acc = acc + jnp.dot(agg_ref[...] * inv, wlt_ref[...],
                        preferred_element_type=jnp.float32)
    if relu:
        acc = jnp.maximum(acc, 0.0)
    out_ref[...] = acc


def _combine(agg, cnt, coff, h, wlt, wrt, b, relu):
    grid = (N // BLK,)
    return pl.pallas_call(
        functools.partial(_combine_body, relu, coff),
        grid=grid,
        in_specs=[
            pl.BlockSpec((BLK, D), lambda i: (i, 0)),
            pl.BlockSpec((BLK, 128), lambda i: (i, 0)),
            pl.BlockSpec((BLK, D), lambda i: (i, 0)),
            pl.BlockSpec((D, D), lambda i: (0, 0)),
            pl.BlockSpec((D, D), lambda i: (0, 0)),
            pl.BlockSpec((1, D), lambda i: (0, 0)),
        ],
        out_specs=pl.BlockSpec((BLK, D), lambda i: (i, 0)),
        out_shape=_f32((N, D)),
    )(agg, cnt, h, wlt, wrt, b)


def _score_body(z1_ref, z2_ref, out_ref):
    z1 = z1_ref[...]
    z2 = z2_ref[...]
    dot = jnp.sum(z1 * z2, axis=1)
    n1 = jnp.maximum(jnp.sqrt(jnp.sum(z1 * z1, axis=1)), 1e-12)
    n2 = jnp.maximum(jnp.sqrt(jnp.sum(z2 * z2, axis=1)), 1e-12)
    out_ref[...] = (dot / (n1 * n2)).reshape(8, 1024)


def _score(z1, z2):
    sb = 8192
    nblk = (EL + sb - 1) // sb
    return pl.pallas_call(
        _score_body,
        grid=(nblk,),
        in_specs=[
            pl.BlockSpec((sb, D), lambda i: (i, 0)),
            pl.BlockSpec((sb, D), lambda i: (i, 0)),
        ],
        out_specs=pl.BlockSpec((8, 1024), lambda i: (i, 0)),
        out_shape=_f32((nblk * 8, 1024)),
    )(z1, z2)


def kernel(x_user, x_recipe, edge_index, edge_label_index,
           W_user, b_user, W_recipe, b_recipe,
           W1_ur_l, W1_ur_r, b1_ur, W1_ru_l, W1_ru_r, b1_ru,
           W2_ur_l, W2_ur_r, b2_ur, W2_ru_l, W2_ru_r, b2_ru):
    src = edge_index[0]
    dst = edge_index[1]
    lsrc = edge_label_index[0]
    ldst = edge_label_index[1]

    pad_g = (jnp.arange(EPAD - E, dtype=jnp.int32) * 37) % N
    pad_s = N + (jnp.arange(EPAD - E, dtype=jnp.int32) % NSENT)
    chunk_off = jnp.arange(NCHUNK, dtype=jnp.int32)[:, None, None]
    src_p = jnp.concatenate([src, pad_g])
    dst_p = jnp.concatenate([dst, pad_g])
    g_ur = src_p.reshape(1, -1, 128) * NCHUNK + chunk_off
    g_ru = dst_p.reshape(1, -1, 128) * NCHUNK + chunk_off
    s_ur = jnp.concatenate([dst, pad_s]).reshape(-1, 128)
    s_ru = jnp.concatenate([src, pad_s]).reshape(-1, 128)
    lpad = (jnp.arange(ELP - EL, dtype=jnp.int32) * 13) % N
    lsrc2 = jnp.concatenate([lsrc, lpad]).reshape(-1, 128)
    ldst2 = jnp.concatenate([ldst, lpad]).reshape(-1, 128)

    zeros32 = jnp.zeros((NROW, CW), jnp.float32)
    ones32 = jnp.ones((128, CW), jnp.float32)

    hu, hr = _encoder(x_user, x_recipe, W_user.T, W_recipe.T,
                      b_user[None, :], b_recipe[None, :])

    agg_r, agg_u, cnt = _segsum_cnt(hu.reshape(-1, CW), hr.reshape(-1, CW),
                                    g_ur, s_ur, g_ru, s_ru, zeros32, ones32)
    r1 = _combine(agg_r, cnt, 0, hr, W1_ur_l.T, W1_ur_r.T, b1_ur[None, :],
                  relu=True)
    u1 = _combine(agg_u, cnt, 32, hu, W1_ru_l.T, W1_ru_r.T, b1_ru[None, :],
                  relu=True)

    agg_r2, agg_u2 = _segsum_plain(u1.reshape(-1, CW), r1.reshape(-1, CW),
                                   g_ur, s_ur, g_ru, s_ru, zeros32)
    r2 = _combine(agg_r2, cnt, 0, r1, W2_ur_l.T, W2_ur_r.T, b2_ur[None, :],
                  relu=False)
    u2 = _combine(agg_u2, cnt, 32, u1, W2_ru_l.T, W2_ru_r.T,
                  b2_ru[None, :], relu=False)

    z1, z2 = _decoder_sc(u2, r2, lsrc2, ldst2)
    scores = _score(z1, z2)
    return scores.reshape(-1)[:EL]


# speedup vs baseline: 1.0368x; 1.0368x over previous
import functools

import jax
import jax.numpy as jnp
from jax import lax
from jax.experimental import pallas as pl
from jax.experimental.pallas import tpu as pltpu
from jax.experimental.pallas import tpu_sc as plsc

NU = 50000
NR = 50000
N = 50000
E = 500000
EL = 100000
D = 128
NCHUNK = 4
CW = 32
NSENT = 48
NROW = N + NSENT

NTILE = 16
EPAD = 524288
EB = 1024
NB = 32
TPE = EB * NB

ELP = 131072
LNB = 4

RPT = NROW // NTILE

BLK = 1000

_mesh = plsc.VectorSubcoreMesh(core_axis_name="c", subcore_axis_name="s")


def _f32(shape):
    return jax.ShapeDtypeStruct(shape, jnp.float32)


def _make_segsum(do_counts):
  out_t = (_f32((NROW, D)), _f32((NROW, D)))
  if do_counts:
    out_t = out_t + (_f32((NROW, 128)),)

  @functools.partial(
      pl.kernel,
      out_type=out_t,
      mesh=_mesh,
      scratch_types=[
          pltpu.VMEM_SHARED((NROW, CW), jnp.float32),
          pltpu.VMEM((32, 128), jnp.int32),
          pltpu.VMEM((32, 128), jnp.int32),
          pltpu.VMEM((512, CW), jnp.float32),
          pltpu.SemaphoreType.DMA,
          pltpu.SemaphoreType.DMA,
      ],
      compiler_params=pltpu.CompilerParams(use_tc_tiling_on_sc=False),
  )
  def _segsum_sc(tbl_r, tbl_u, g_ur, s_ur, g_ru, s_ru, zeros_hbm, *rest):
    if do_counts:
        ones_hbm, agg_r, agg_u, cnt_out = rest[:4]
        acc, gidxb, sidxb, rows_v, semg, sems = rest[4:]
    else:
        agg_r, agg_u = rest[:2]
        acc, gidxb, sidxb, rows_v, semg, sems = rest[2:]
    cid = lax.axis_index("c")
    sid = lax.axis_index("s")
    base_r = sid * (TPE // 128)

    def region(s):
        return rows_v.at[pl.ds((s % 4) * 128, 128)]

    for direction in range(2):
        tbl = (tbl_r, tbl_u)[direction]
        gi = (g_ur, g_ru)[direction]
        si = (s_ur, s_ru)[direction]
        out = (agg_r, agg_u)[direction]
        for j in range(2):
            chunk = cid * 2 + j
            gic = gi.at[chunk]
            pltpu.sync_copy(zeros_hbm.at[pl.ds(sid * RPT, RPT)],
                            acc.at[pl.ds(sid * RPT, RPT)])
            plsc.subcore_barrier()

            def batch(b, carry):
                roff = base_r + b * 32
                pltpu.sync_copy(gic.at[pl.ds(roff, 32)], gidxb)
                pltpu.sync_copy(si.at[pl.ds(roff, 32)], sidxb)
                gat = [None] * 32
                scat = [None] * 32
                for t in range(32):
                    if t >= 4:
                        scat[t - 4].wait()
                    else:
                        @pl.when(b > 0)
                        def _(t=t):
                            pltpu.make_async_copy(
                                region(t), acc.at[sidxb.at[t]], sems).wait()
                    gat[t] = pltpu.async_copy(tbl.at[gidxb.at[t]],
                                              region(t), semg)
                    if t >= 2:
                        gat[t - 2].wait()
                        scat[t - 2] = pltpu.async_copy(
                            region(t - 2), acc.at[sidxb.at[t - 2]], sems,
                            add=True)
                for t in (30, 31):
                    gat[t].wait()
                    scat[t] = pltpu.async_copy(region(t), acc.at[sidxb.at[t]],
                                               sems, add=True)
                return carry

            lax.fori_loop(0, NB // 4, batch, 0)
            for t in range(4):
                pltpu.make_async_copy(region(t), acc.at[sidxb.at[t]],
                                      sems).wait()
            plsc.subcore_barrier()
            pltpu.sync_copy(acc.at[pl.ds(sid * RPT, RPT)],
                            out.at[pl.ds(sid * RPT, RPT),
                                   pl.ds(chunk * CW, CW)])
            plsc.subcore_barrier()

    if do_counts:
        pltpu.sync_copy(ones_hbm, rows_v.at[pl.ds(0, 128)])
        pltpu.sync_copy(zeros_hbm.at[pl.ds(sid * RPT, RPT)],
                        acc.at[pl.ds(sid * RPT, RPT)])
        plsc.subcore_barrier()

        def hist(si):
            def hblk(b, carry):
                pltpu.sync_copy(si.at[pl.ds(base_r + b * 32, 32)], sidxb)
                for g in range(4):
                    scs = [
                        pltpu.async_copy(rows_v.at[pl.ds(0, 128)],
                                         acc.at[sidxb.at[g * 8 + t]], sems,
                                         add=True)
                        for t in range(8)
                    ]
                    for dsc in scs:
                        dsc.wait()
                return carry
            lax.fori_loop(0, NB // 4, hblk, 0)

        @pl.when(cid == 0)
        def _():
            hist(s_ur)

        @pl.when(cid == 1)
        def _():
            hist(s_ru)

        plsc.subcore_barrier()
        pltpu.sync_copy(acc.at[pl.ds(sid * RPT, RPT)],
                        cnt_out.at[pl.ds(sid * RPT, RPT),
                                   pl.ds(cid * 32, 32)])

  return _segsum_sc


_segsum_plain = _make_segsum(False)


@functools.partial(
    pl.kernel,
    out_type=_f32((NROW, 128)),
    mesh=_mesh,
    scratch_types=[
        pltpu.VMEM_SHARED((NROW, 16), jnp.float32),
        pltpu.VMEM((8, 128), jnp.int32),
        pltpu.VMEM((128, 16), jnp.float32),
    ],
    compiler_params=pltpu.CompilerParams(use_tc_tiling_on_sc=False),
)
def _counts_sc(s_ur, s_ru, zeros16_hbm, ones_hbm,
               cnt_out, acc, sidx_v, ones_v):
    cid = lax.axis_index("c")
    sid = lax.axis_index("s")
    base_r = sid * (TPE // 128)
    pltpu.sync_copy(ones_hbm, ones_v)
    pltpu.sync_copy(zeros16_hbm.at[pl.ds(sid * RPT, RPT)],
                    acc.at[pl.ds(sid * RPT, RPT)])
    plsc.subcore_barrier()

    def _hist(si, coff):
        def batch(b, _):
            pltpu.sync_copy(si.at[pl.ds(base_r + b * 8, 8)], sidx_v)
            for r in range(8):
                pltpu.sync_copy(ones_v, acc.at[sidx_v.at[r]], add=True)
            return _
        lax.fori_loop(0, NB, batch, 0)
        plsc.subcore_barrier()
        pltpu.sync_copy(acc.at[pl.ds(sid * RPT, RPT)],
                        cnt_out.at[pl.ds(sid * RPT, RPT), pl.ds(coff, 16)])

    @pl.when(cid == 0)
    def _():
        _hist(s_ur, 0)

    @pl.when(cid == 1)
    def _():
        _hist(s_ru, 16)


@functools.partial(
    pl.kernel,
    out_type=(_f32((ELP, D)), _f32((ELP, D))),
    mesh=_mesh,
    scratch_types=[
        pltpu.VMEM((8, 128), jnp.int32),
        pltpu.VMEM((512, D), jnp.float32),
        pltpu.SemaphoreType.DMA,
        pltpu.SemaphoreType.DMA,
    ],
    compiler_params=pltpu.CompilerParams(use_tc_tiling_on_sc=True),
)
def _decoder_sc(u2, r2, lsrc2, ldst2, z1, z2, idx_v, rows_v, semg, semo):
    cid = lax.axis_index("c")
    sid = lax.axis_index("s")
    wid = sid * 2 + cid
    per_tile = ELP // 32

    def buf(u):
        return rows_v.at[pl.ds((u % 2) * 256, 256)]

    for direction in range(2):
        tbl = (u2, r2)[direction]
        li = (lsrc2, ldst2)[direction]
        out = (z1, z2)[direction]

        def batch(b, carry):
            roff = wid * (per_tile // 128) + b * 8
            pltpu.sync_copy(li.at[pl.ds(roff, 8)], idx_v)
            ocp = [None] * 4
            for u in range(4):
                obase = wid * per_tile + b * 1024 + u * 256
                if u >= 2:
                    ocp[u - 2].wait()
                else:
                    @pl.when(b > 0)
                    def _(u=u, obase=obase):
                        pltpu.make_async_copy(
                            buf(u), out.at[pl.ds(obase, 256)], semo).wait()
                gats = [
                    pltpu.async_copy(tbl.at[idx_v.at[u * 2 + q]],
                                     buf(u).at[pl.ds(q * 128, 128)], semg)
                    for q in range(2)
                ]
                for dsc in gats:
                    dsc.wait()
                ocp[u] = pltpu.async_copy(buf(u), out.at[pl.ds(obase, 256)],
                                          semo)
            return carry

        lax.fori_loop(0, LNB, batch, 0)
        for u in range(2):
            pltpu.make_async_copy(buf(u), out.at[pl.ds(wid * per_tile, 256)],
                                  semo).wait()


def _enc_body(xu_ref, xr_ref, wut_ref, wrt_ref, bu_ref, br_ref,
              hu_ref, hr_ref):
    hu_ref[...] = jnp.dot(xu_ref[...], wut_ref[...],
                          preferred_element_type=jnp.float32) + bu_ref[...]
    hr_ref[...] = jnp.dot(xr_ref[...], wrt_ref[...],
                          preferred_element_type=jnp.float32) + br_ref[...]


def _encoder(xu, xr, wut, wrt, bu, br):
    grid = (N // BLK,)
    row = pl.BlockSpec((BLK, D), lambda i: (i, 0))
    full = pl.BlockSpec((D, D), lambda i: (0, 0))
    bias = pl.BlockSpec((1, D), lambda i: (0, 0))
    return pl.pallas_call(
        _enc_body,
        grid=grid,
        in_specs=[row, row, full, full, bias, bias],
        out_specs=[row, row],
        out_shape=[_f32((N, D)), _f32((N, D))],
    )(xu, xr, wut, wrt, bu, br)


def _combine_body(relu, coff, agg_ref, cnt_ref, h_ref, wlt_ref, wrt_ref,
                  b_ref, out_ref):
    inv = 1.0 / jnp.maximum(cnt_ref[:, coff:coff + 1], 1.0)
    acc = jnp.dot(h_ref[...], wrt_ref[...],
                  preferred_element_type=jnp.float32) + b_ref[...]
    acc = acc + jnp.dot(agg_ref[...] * inv, wlt_ref[...],
                        preferred_element_type=jnp.float32)
    if relu:
        acc = jnp.maximum(acc, 0.0)
    out_ref[...] = acc


def _combine(agg, cnt, coff, h, wlt, wrt, b, relu):
    grid = (N // BLK,)
    return pl.pallas_call(
        functools.partial(_combine_body, relu, coff),
        grid=grid,
        in_specs=[
            pl.BlockSpec((BLK, D), lambda i: (i, 0)),
            pl.BlockSpec((BLK, 128), lambda i: (i, 0)),
            pl.BlockSpec((BLK, D), lambda i: (i, 0)),
            pl.BlockSpec((D, D), lambda i: (0, 0)),
            pl.BlockSpec((D, D), lambda i: (0, 0)),
            pl.BlockSpec((1, D), lambda i: (0, 0)),
        ],
        out_specs=pl.BlockSpec((BLK, D), lambda i: (i, 0)),
        out_shape=_f32((N, D)),
    )(agg, cnt, h, wlt, wrt, b)


def _score_body(z1_ref, z2_ref, out_ref):
    z1 = z1_ref[...]
    z2 = z2_ref[...]
    dot = jnp.sum(z1 * z2, axis=1)
    n1 = jnp.maximum(jnp.sqrt(jnp.sum(z1 * z1, axis=1)), 1e-12)
    n2 = jnp.maximum(jnp.sqrt(jnp.sum(z2 * z2, axis=1)), 1e-12)
    out_ref[...] = (dot / (n1 * n2)).reshape(8, 1024)


def _score(z1, z2):
    sb = 8192
    nblk = (EL + sb - 1) // sb
    return pl.pallas_call(
        _score_body,
        grid=(nblk,),
        in_specs=[
            pl.BlockSpec((sb, D), lambda i: (i, 0)),
            pl.BlockSpec((sb, D), lambda i: (i, 0)),
        ],
        out_specs=pl.BlockSpec((8, 1024), lambda i: (i, 0)),
        out_shape=_f32((nblk * 8, 1024)),
    )(z1, z2)


def kernel(x_user, x_recipe, edge_index, edge_label_index,
           W_user, b_user, W_recipe, b_recipe,
           W1_ur_l, W1_ur_r, b1_ur, W1_ru_l, W1_ru_r, b1_ru,
           W2_ur_l, W2_ur_r, b2_ur, W2_ru_l, W2_ru_r, b2_ru):
    src = edge_index[0]
    dst = edge_index[1]
    lsrc = edge_label_index[0]
    ldst = edge_label_index[1]

    pad_g = (jnp.arange(EPAD - E, dtype=jnp.int32) * 37) % N
    pad_s = N + (jnp.arange(EPAD - E, dtype=jnp.int32) % NSENT)
    chunk_off = jnp.arange(NCHUNK, dtype=jnp.int32)[:, None, None]
    src_p = jnp.concatenate([src, pad_g])
    dst_p = jnp.concatenate([dst, pad_g])
    g_ur = src_p.reshape(1, -1, 128) * NCHUNK + chunk_off
    g_ru = dst_p.reshape(1, -1, 128) * NCHUNK + chunk_off
    s_ur = jnp.concatenate([dst, pad_s]).reshape(-1, 128)
    s_ru = jnp.concatenate([src, pad_s]).reshape(-1, 128)
    lpad = (jnp.arange(ELP - EL, dtype=jnp.int32) * 13) % N
    lsrc2 = jnp.concatenate([lsrc, lpad]).reshape(-1, 128)
    ldst2 = jnp.concatenate([ldst, lpad]).reshape(-1, 128)

    zeros32 = jnp.zeros((NROW, CW), jnp.float32)
    zeros16 = zeros32.reshape(-1, 16)[:NROW]
    ones16 = jnp.ones((128, 16), jnp.float32)

    hu, hr = _encoder(x_user, x_recipe, W_user.T, W_recipe.T,
                      b_user[None, :], b_recipe[None, :])
    cnt = _counts_sc(s_ur, s_ru, zeros16, ones16)

    agg_r, agg_u = _segsum_plain(hu.reshape(-1, CW), hr.reshape(-1, CW),
                                 g_ur, s_ur, g_ru, s_ru, zeros32)
    r1 = _combine(agg_r, cnt, 0, hr, W1_ur_l.T, W1_ur_r.T, b1_ur[None, :],
                  relu=True)
    u1 = _combine(agg_u, cnt, 16, hu, W1_ru_l.T, W1_ru_r.T, b1_ru[None, :],
                  relu=True)

    agg_r2, agg_u2 = _segsum_plain(u1.reshape(-1, CW), r1.reshape(-1, CW),
                                   g_ur, s_ur, g_ru, s_ru, zeros32)
    r2 = _combine(agg_r2, cnt, 0, r1, W2_ur_l.T, W2_ur_r.T, b2_ur[None, :],
                  relu=False)
    u2 = _combine(agg_u2, cnt, 16, u1, W2_ru_l.T, W2_ru_r.T,
                  b2_ru[None, :], relu=False)

    z1, z2 = _decoder_sc(u2, r2, lsrc2, ldst2)
    scores = _score(z1, z2)
    return scores.reshape(-1)[:EL]
